# baseline (speedup 1.0000x reference)
import functools

import jax
import jax.numpy as jnp
import numpy as np
from jax import lax
from jax.experimental import pallas as pl
from jax.experimental.pallas import tpu as pltpu
from jax.experimental.pallas import tpu_sc as plsc

VOCAB = 30522
DIM = 768
MAX_LEN = 64
BATCH = 4096
SEQ = 64

NUM_CORES = 2
NUM_SUBCORES = 16
NUM_WORKERS = NUM_CORES * NUM_SUBCORES
S_PER_WORKER = SEQ // NUM_WORKERS
ROWS_PER_WORKER = S_PER_WORKER * BATCH
CHUNK = 32
CHUNKS_PER_S = BATCH // CHUNK
NCHUNKS = S_PER_WORKER * CHUNKS_PER_S
RING = 4
LANES = 16
COL_GROUPS = DIM // LANES


def _fixed_sincos1d(length, dim):
    pos = np.arange(length, dtype=np.float32)[:, None]
    i = np.arange(dim // 2, dtype=np.float32)[None, :]
    angle = pos / np.power(10000.0, 2.0 * i / dim)
    return np.concatenate([np.sin(angle), np.cos(angle)], axis=-1)


def _embed_kernel(xt_hbm, wte_hbm, pos_hbm, out_hbm,
                  idx_all, buf_v, pos_v, gsem, ssem):
    wid = lax.axis_index("s") * NUM_CORES + lax.axis_index("c")
    s_base = wid * S_PER_WORKER

    pltpu.sync_copy(xt_hbm.at[pl.ds(wid * ROWS_PER_WORKER, ROWS_PER_WORKER)],
                    idx_all)
    pltpu.sync_copy(pos_hbm.at[pl.ds(s_base, S_PER_WORKER)], pos_v)

    def fill(q, par):
        @pl.when(q >= RING)
        def _():
            pltpu.make_async_copy(
                buf_v.at[par], out_hbm.at[pl.ds(0, CHUNK), 0], ssem.at[par]
            ).wait()

        pltpu.make_async_copy(
            wte_hbm.at[idx_all.at[pl.ds(q * CHUNK, CHUNK)]],
            buf_v.at[par],
            gsem.at[par],
        ).start()

    def drain(q, par):
        pltpu.make_async_copy(
            wte_hbm.at[pl.ds(0, CHUNK)], buf_v.at[par], gsem.at[par]
        ).wait()
        t = q // CHUNKS_PER_S
        b0 = (q % CHUNKS_PER_S) * CHUNK

        for c in range(COL_GROUPS):
            sl = pl.ds(c * LANES, LANES)
            pv = pos_v[t, sl]

            @plsc.parallel_loop(0, CHUNK, unroll=16)
            def _(r):
                buf_v[par, r, sl] = buf_v[par, r, sl] + pv

        pltpu.make_async_copy(
            buf_v.at[par], out_hbm.at[pl.ds(b0, CHUNK), s_base + t],
            ssem.at[par],
        ).start()

    for p in range(RING - 1):
        fill(p, p)

    def body(k, carry):
        q0 = RING * k
        for j in range(RING):
            q = q0 + j
            drain(q, j)

            @pl.when(q + RING - 1 < NCHUNKS)
            def _():
                fill(q + RING - 1, (j + RING - 1) % RING)

        return carry

    lax.fori_loop(0, NCHUNKS // RING, body, 0)

    for par in range(RING):
        pltpu.make_async_copy(
            buf_v.at[par], out_hbm.at[pl.ds(0, CHUNK), 0], ssem.at[par]
        ).wait()


@functools.partial(jax.jit, static_argnames=())
def kernel(x, wte):
    pos = jnp.asarray(_fixed_sincos1d(MAX_LEN, DIM), dtype=jnp.float32)
    xt_flat = x.astype(jnp.int32).T.reshape(SEQ * BATCH)

    mesh = plsc.VectorSubcoreMesh(core_axis_name="c", subcore_axis_name="s")
    run = pl.kernel(
        _embed_kernel,
        mesh=mesh,
        out_type=jax.ShapeDtypeStruct((BATCH, SEQ, DIM), jnp.float32),
        scratch_types=[
            pltpu.VMEM((ROWS_PER_WORKER,), jnp.int32),
            pltpu.VMEM((RING, CHUNK, DIM), jnp.float32),
            pltpu.VMEM((S_PER_WORKER, DIM), jnp.float32),
            pltpu.SemaphoreType.DMA((RING,)),
            pltpu.SemaphoreType.DMA((RING,)),
        ],
    )
    return run(xt_flat, wte, pos)


# speedup vs baseline: 1.1331x; 1.1331x over previous
import functools

import jax
import jax.numpy as jnp
import numpy as np
from jax import lax
from jax.experimental import pallas as pl
from jax.experimental.pallas import tpu as pltpu
from jax.experimental.pallas import tpu_sc as plsc

VOCAB = 30522
DIM = 768
MAX_LEN = 64
BATCH = 4096
SEQ = 64

NUM_CORES = 2
NUM_SUBCORES = 16
NUM_WORKERS = NUM_CORES * NUM_SUBCORES
S_PER_WORKER = SEQ // NUM_WORKERS
ROWS_PER_WORKER = S_PER_WORKER * BATCH
CHUNK = 32
CHUNKS_PER_S = BATCH // CHUNK
NCHUNKS = S_PER_WORKER * CHUNKS_PER_S
RING = 4
LANES = 16
COL_GROUPS = DIM // LANES


def _fixed_sincos1d(length, dim):
    pos = np.arange(length, dtype=np.float32)[:, None]
    i = np.arange(dim // 2, dtype=np.float32)[None, :]
    angle = pos / np.power(10000.0, 2.0 * i / dim)
    return np.concatenate([np.sin(angle), np.cos(angle)], axis=-1)


def _embed_kernel(xt_hbm, wte_hbm, pos_hbm, out_hbm,
                  idx_all, buf_v, pos_v, gsem, ssem):
    wid = lax.axis_index("s") * NUM_CORES + lax.axis_index("c")
    s_base = wid * S_PER_WORKER

    pltpu.sync_copy(xt_hbm.at[pl.ds(wid * ROWS_PER_WORKER, ROWS_PER_WORKER)],
                    idx_all)
    pltpu.sync_copy(pos_hbm.at[pl.ds(s_base, S_PER_WORKER)], pos_v)

    def fill(q, par):
        @pl.when(q >= RING)
        def _():
            pltpu.make_async_copy(
                buf_v.at[par], out_hbm.at[pl.ds(0, CHUNK), 0], ssem.at[par]
            ).wait()

        pltpu.make_async_copy(
            wte_hbm.at[idx_all.at[pl.ds(q * CHUNK, CHUNK)]],
            buf_v.at[par],
            gsem.at[par],
        ).start()

    def drain(q, par):
        pltpu.make_async_copy(
            wte_hbm.at[pl.ds(0, CHUNK)], buf_v.at[par], gsem.at[par]
        ).wait()
        t = q // CHUNKS_PER_S
        b0 = (q % CHUNKS_PER_S) * CHUNK

        pvs = [pos_v[t, pl.ds(c * LANES, LANES)] for c in range(COL_GROUPS)]

        @plsc.parallel_loop(0, CHUNK, unroll=2)
        def _(r):
            for c in range(COL_GROUPS):
                sl = pl.ds(c * LANES, LANES)
                buf_v[par, r, sl] = buf_v[par, r, sl] + pvs[c]

        pltpu.make_async_copy(
            buf_v.at[par], out_hbm.at[pl.ds(b0, CHUNK), s_base + t],
            ssem.at[par],
        ).start()

    for p in range(RING - 1):
        fill(p, p)

    def body(k, carry):
        q0 = RING * k
        for j in range(RING):
            q = q0 + j
            drain(q, j)

            @pl.when(q + RING - 1 < NCHUNKS)
            def _():
                fill(q + RING - 1, (j + RING - 1) % RING)

        return carry

    lax.fori_loop(0, NCHUNKS // RING, body, 0)

    for par in range(RING):
        pltpu.make_async_copy(
            buf_v.at[par], out_hbm.at[pl.ds(0, CHUNK), 0], ssem.at[par]
        ).wait()


@functools.partial(jax.jit, static_argnames=())
def kernel(x, wte):
    pos = jnp.asarray(_fixed_sincos1d(MAX_LEN, DIM), dtype=jnp.float32)
    xt_flat = x.astype(jnp.int32).T.reshape(SEQ * BATCH)

    mesh = plsc.VectorSubcoreMesh(core_axis_name="c", subcore_axis_name="s")
    run = pl.kernel(
        _embed_kernel,
        mesh=mesh,
        out_type=jax.ShapeDtypeStruct((BATCH, SEQ, DIM), jnp.float32),
        scratch_types=[
            pltpu.VMEM((ROWS_PER_WORKER,), jnp.int32),
            pltpu.VMEM((RING, CHUNK, DIM), jnp.float32),
            pltpu.VMEM((S_PER_WORKER, DIM), jnp.float32),
            pltpu.SemaphoreType.DMA((RING,)),
            pltpu.SemaphoreType.DMA((RING,)),
        ],
    )
    return run(xt_flat, wte, pos)
